# baseline (speedup 1.0000x reference)
import functools

import jax
import jax.numpy as jnp
from jax import lax
from jax.experimental import pallas as pl
from jax.experimental.pallas import tpu as pltpu
from jax.experimental.pallas import tpu_sc as plsc

_ROWS = 16384
_HID = 2048
_EXPERTS = 64
_BR = 512
_NW = 32
_L = 16
_NCHUNK = 2
_CROWS = _ROWS // _NCHUNK
_WBR = _CROWS // _NW
_GROUPS = _WBR // _L


def _logits_kernel(x_ref, w_ref, out_ref):
    out_ref[0] = jax.lax.dot_general(
        w_ref[...], x_ref[...], (((1,), (1,)), ((), ())),
        preferred_element_type=jnp.float32,
    )


def _tc_logits(x, w):
    return pl.pallas_call(
        _logits_kernel,
        grid=(_CROWS // _BR,),
        in_specs=[
            pl.BlockSpec((_BR, _HID), lambda i: (i, 0)),
            pl.BlockSpec((_EXPERTS, _HID), lambda i: (0, 0)),
        ],
        out_specs=pl.BlockSpec((1, _EXPERTS, _BR), lambda i: (i, 0, 0)),
        out_shape=jax.ShapeDtypeStruct((_CROWS // _BR, _EXPERTS, _BR), jnp.float32),
    )(x, w)


def _sc_topk(logits_blk):
    mesh = plsc.VectorSubcoreMesh(core_axis_name="c", subcore_axis_name="s")

    @functools.partial(
        pl.kernel,
        mesh=mesh,
        out_type=[
            jax.ShapeDtypeStruct((2 * _CROWS,), jnp.float32),
            jax.ShapeDtypeStruct((2 * _CROWS,), jnp.int32),
        ],
        scratch_types=[
            pltpu.VMEM((_EXPERTS, _WBR), jnp.float32),
            pltpu.VMEM((2 * _WBR,), jnp.float32),
            pltpu.VMEM((2 * _WBR,), jnp.int32),
        ],
        compiler_params=pltpu.CompilerParams(needs_layout_passes=False),
    )
    def body(lg_hbm, val_hbm, idx_hbm, lg_v, val_v, idx_v):
        wid = lax.axis_index("s") * 2 + lax.axis_index("c")
        nblk = _CROWS // _BR
        blk = wid * _WBR // _BR
        col = wid * _WBR % _BR
        pltpu.sync_copy(lg_hbm.at[blk, :, pl.ds(col, _WBR)], lg_v)

        def group(g, _):
            neg = jnp.full((_L,), -jnp.inf, jnp.float32)
            zero = jnp.zeros((_L,), jnp.int32)
            m1, i1, m2, i2 = neg, zero, neg, zero
            for e in range(_EXPERTS):
                v = lg_v[e, pl.ds(g * _L, _L)]
                es = jnp.full((_L,), e, jnp.int32)
                gt1 = v > m1
                gt2 = v > m2
                m2 = jnp.where(gt1, m1, jnp.where(gt2, v, m2))
                i2 = jnp.where(gt1, i1, jnp.where(gt2, es, i2))
                m1 = jnp.where(gt1, v, m1)
                i1 = jnp.where(gt1, es, i1)
            w2 = jnp.exp(m2 - m1)
            inv = 1.0 / (1.0 + w2)
            pos = g * (2 * _L) + 2 * lax.iota(jnp.int32, _L)
            plsc.store_scatter(val_v, [pos], inv)
            plsc.store_scatter(val_v, [pos + 1], w2 * inv)
            plsc.store_scatter(idx_v, [pos], i1)
            plsc.store_scatter(idx_v, [pos + 1], i2)
            return ()

        lax.fori_loop(0, _GROUPS, group, (), unroll=False)
        base = wid * (2 * _WBR)
        pltpu.sync_copy(val_v, val_hbm.at[pl.ds(base, 2 * _WBR)])
        pltpu.sync_copy(idx_v, idx_hbm.at[pl.ds(base, 2 * _WBR)])

    return body(logits_blk)


@jax.jit
def kernel(hidden_states, weight):
    vals_parts, idx_parts = [], []
    for c in range(_NCHUNK):
        x = lax.slice_in_dim(hidden_states, c * _CROWS, (c + 1) * _CROWS, axis=0)
        lg = _tc_logits(x, weight)
        v, i = _sc_topk(lg)
        vals_parts.append(v.reshape(_CROWS, 2))
        idx_parts.append(i.reshape(_CROWS, 2))
    return (
        jnp.concatenate(vals_parts, axis=0),
        jnp.concatenate(idx_parts, axis=0),
    )


# speedup vs baseline: 4.0493x; 4.0493x over previous
import jax
import jax.numpy as jnp
from jax.experimental import pallas as pl

_ROWS = 16384
_HID = 2048
_EXPERTS = 64
_BR = 1024


def _router_kernel(x_ref, w_ref, val_ref, idx_ref):
    lg = jax.lax.dot_general(
        w_ref[...], x_ref[...], (((1,), (1,)), ((), ())),
        preferred_element_type=jnp.float32,
    )
    iota = jax.lax.broadcasted_iota(jnp.int32, lg.shape, 0)
    m1 = jnp.max(lg, axis=0, keepdims=True)
    i1 = jnp.min(jnp.where(lg == m1, iota, _EXPERTS), axis=0, keepdims=True)
    masked = jnp.where(iota == i1, -jnp.inf, lg)
    m2 = jnp.max(masked, axis=0, keepdims=True)
    i2 = jnp.min(jnp.where(masked == m2, iota, _EXPERTS), axis=0, keepdims=True)
    e2 = jnp.exp(m2 - m1)
    inv = 1.0 / (1.0 + e2)
    val_ref[...] = jnp.concatenate([inv, e2 * inv], axis=0)
    idx_ref[...] = jnp.concatenate([i1, i2], axis=0)


@jax.jit
def kernel(hidden_states, weight):
    grid = (_ROWS // _BR,)
    vals, idx = pl.pallas_call(
        _router_kernel,
        grid=grid,
        in_specs=[
            pl.BlockSpec((_BR, _HID), lambda i: (i, 0)),
            pl.BlockSpec((_EXPERTS, _HID), lambda i: (0, 0)),
        ],
        out_specs=[
            pl.BlockSpec((2, _BR), lambda i: (0, i)),
            pl.BlockSpec((2, _BR), lambda i: (0, i)),
        ],
        out_shape=[
            jax.ShapeDtypeStruct((2, _ROWS), jnp.float32),
            jax.ShapeDtypeStruct((2, _ROWS), jnp.int32),
        ],
    )(hidden_states, weight)
    return (vals.T, idx.T)
